# baseline (speedup 1.0000x reference)
import functools

import jax
import jax.numpy as jnp
import numpy as np
from jax import lax
from jax.experimental import pallas as pl
from jax.experimental.pallas import tpu as pltpu
from jax.experimental.pallas import tpu_sc as plsc

_VOCAB = 1000000
_EMB = 64
_L = 50
_RAD = 5
_NEG = 5
_B = 4096

_NC = 2
_NS = 16
_NW = _NC * _NS
_POS_ROWS = _B * _L
_NEG_ROWS = _B * _NEG
_ROWS = _POS_ROWS + _NEG_ROWS
_RPW = _ROWS // _NW
_CH = 128
_NCH = _RPW // _CH

_BSUB = 2048
_NB2 = _B // _BSUB
_NPROD = 11
_HALF = 123 * 4096
_PACKED_ROWS = _POS_ROWS // 2 + _NEG_ROWS
_T = _L // 2


def _sc_gather(table2, idxq3):
    mesh = plsc.VectorSubcoreMesh(core_axis_name="c", subcore_axis_name="s")

    @functools.partial(
        pl.kernel,
        mesh=mesh,
        out_type=jax.ShapeDtypeStruct((_ROWS, 2 * _EMB), jnp.float32),
        scratch_types=[
            pltpu.VMEM((_NCH, _CH), jnp.int32),
            pltpu.VMEM((_CH, 2 * _EMB), jnp.float32),
            pltpu.VMEM((_CH, 2 * _EMB), jnp.float32),
            pltpu.SemaphoreType.DMA,
            pltpu.SemaphoreType.DMA,
        ],
    )
    def gather_kernel(table_hbm, idx_hbm, out_hbm, idx_v, rows_a, rows_b,
                      sem_a, sem_b):
        wid = lax.axis_index("s") * _NC + lax.axis_index("c")
        pltpu.sync_copy(idx_hbm.at[wid], idx_v)
        base = wid * _RPW

        def start(j, buf, sem):
            pltpu.async_copy(table_hbm.at[idx_v.at[j]], buf, sem)

        def finish(j, buf, sem):
            pltpu.make_async_copy(table_hbm.at[idx_v.at[j]], buf, sem).wait()
            pltpu.sync_copy(buf, out_hbm.at[pl.ds(base + j * _CH, _CH)])

        start(0, rows_a, sem_a)

        def body(m, carry):
            j0 = 2 * m
            start(j0 + 1, rows_b, sem_b)
            finish(j0, rows_a, sem_a)
            start(jnp.minimum(j0 + 2, _NCH - 1), rows_a, sem_a)
            finish(j0 + 1, rows_b, sem_b)
            return carry

        lax.fori_loop(0, _NCH // 2, body, 0)
        finish(_NCH - 1, rows_a, sem_a)

    return gather_kernel(table2, idxq3)


def _tc_squeeze(tT):
    bc = 6144
    grid = _HALF // bc

    def body(l_ref, r_ref, o_ref):
        o_ref[:, :_EMB] = jnp.transpose(l_ref[...], (1, 0))
        o_ref[:, _EMB:] = jnp.transpose(r_ref[...], (1, 0))

    return pl.pallas_call(
        body,
        grid=(grid,),
        in_specs=[
            pl.BlockSpec((_EMB, bc), lambda i: (0, i)),
            pl.BlockSpec((_EMB, bc),
                         lambda i: (0, jnp.minimum(i + grid,
                                                   _VOCAB // bc))),
        ],
        out_specs=pl.BlockSpec((bc, 2 * _EMB), lambda i: (i, 0)),
        out_shape=jax.ShapeDtypeStruct((_HALF, 2 * _EMB), jnp.float32),
    )(tT, tT)


def _tc_unpack(packed, hsel3):
    blk = _B
    npos = _L // 2

    def i1(o):
        return (jnp.where(o < npos, 2 * o, npos + o), 0)

    def i2(o):
        return (jnp.where(o < npos, 2 * o + 1, npos + o), 0)

    def body(p1_ref, p2_ref, h1_ref, h2_ref, o_ref):
        p1, p2 = p1_ref[...], p2_ref[...]
        h1 = jnp.transpose(h1_ref[...].reshape(1, blk), (1, 0))
        h2 = jnp.transpose(h2_ref[...].reshape(1, blk), (1, 0))
        o_ref[:, :_EMB] = jnp.where(h1 > 0, p1[:, _EMB:], p1[:, :_EMB])
        o_ref[:, _EMB:] = jnp.where(h2 > 0, p2[:, _EMB:], p2[:, :_EMB])

    return pl.pallas_call(
        body,
        grid=(_ROWS // blk // 2 + _NEG,),
        in_specs=[
            pl.BlockSpec((blk, 2 * _EMB), i1),
            pl.BlockSpec((blk, 2 * _EMB), i2),
            pl.BlockSpec((1, 1, blk), lambda o: (i1(o)[0], 0, 0)),
            pl.BlockSpec((1, 1, blk), lambda o: (i2(o)[0], 0, 0)),
        ],
        out_specs=pl.BlockSpec((blk, 2 * _EMB), lambda o: (o, 0)),
        out_shape=jax.ShapeDtypeStruct((_PACKED_ROWS, 2 * _EMB), jnp.float32),
    )(packed, packed, hsel3, hsel3)


def _sel_matrix():
    sel = np.zeros((_NPROD * 128, 128), np.float32)
    for p in range(_NPROD):
        sel[p * 128: p * 128 + 64, 2 * p] = 1.0
        sel[p * 128 + 64: (p + 1) * 128, 2 * p + 1] = 1.0
    return jnp.asarray(sel)


_POS_TMAX = [23, 23, 22, 22, 24, -1, 23, 23, 22, 22, -1, 21]
_NCOL = 2 * _NPROD


def _tc_loss(g2):
    negblk0 = _POS_ROWS // 2 // _BSUB

    def body(a_ref, b1_ref, b2_ref, b3_ref, n0, n1, n2, n3, n4, sel_ref,
             out_ref, s_ref):
        i2 = pl.program_id(0)
        t = pl.program_id(1)
        a = a_ref[...]
        prods = [
            a * b1_ref[...],
            a * b2_ref[...],
            a * pltpu.roll(a, 64, 1),
            a * pltpu.roll(b1_ref[...], 64, 1),
            a * pltpu.roll(b2_ref[...], 64, 1),
            a * pltpu.roll(b3_ref[...], 64, 1),
            a * n0[...],
            a * n1[...],
            a * n2[...],
            a * n3[...],
            a * n4[...],
        ]
        for p in range(_NPROD):
            s_ref[:, p * 128:(p + 1) * 128] = prods[p]
        sims = jnp.dot(s_ref[...], sel_ref[...],
                       preferred_element_type=jnp.float32)

        p_ = jax.nn.sigmoid(sims)
        f = jnp.where(p_ > 0, -jnp.log(jnp.where(p_ > 0, p_, 1.0)), 100.0)
        q_ = 1.0 - p_
        g = jnp.where(q_ > 0, -jnp.log(jnp.where(q_ > 0, q_, 1.0)), 100.0)

        lanes = lax.broadcasted_iota(jnp.int32, (1, 128), 1)
        tmax = jnp.full((1, 128), -1, jnp.int32)
        for c, tm in enumerate(_POS_TMAX):
            tmax = jnp.where(lanes == c, tm, tmax)
        is_pos = lanes < 12
        is_neg = (lanes >= 12) & (lanes < _NCOL)
        w = jnp.where(is_pos & (t <= tmax), 2.0,
                      jnp.where(is_neg, 1.0, 0.0))
        vals = jnp.where(is_pos, f, g) * w
        part = jnp.sum(vals, axis=0, keepdims=True)

        @pl.when((i2 == 0) & (t == 0))
        def _():
            out_ref[...] = jnp.zeros_like(out_ref)

        out_ref[...] += part

    bspec = lambda im: pl.BlockSpec((_BSUB, 128), im)
    out = pl.pallas_call(
        body,
        grid=(_NB2, _T),
        in_specs=[
            bspec(lambda i2, t: (t * _NB2 + i2, 0)),
            bspec(lambda i2, t: (jnp.minimum(t + 1, _T - 1) * _NB2 + i2, 0)),
            bspec(lambda i2, t: (jnp.minimum(t + 2, _T - 1) * _NB2 + i2, 0)),
            bspec(lambda i2, t: (jnp.minimum(t + 3, _T - 1) * _NB2 + i2, 0)),
            bspec(lambda i2, t: (negblk0 + 0 * _NB2 + i2, 0)),
            bspec(lambda i2, t: (negblk0 + 1 * _NB2 + i2, 0)),
            bspec(lambda i2, t: (negblk0 + 2 * _NB2 + i2, 0)),
            bspec(lambda i2, t: (negblk0 + 3 * _NB2 + i2, 0)),
            bspec(lambda i2, t: (negblk0 + 4 * _NB2 + i2, 0)),
            pl.BlockSpec((_NPROD * 128, 128), lambda i2, t: (0, 0)),
        ],
        out_specs=pl.BlockSpec((1, 128), lambda i2, t: (0, 0)),
        out_shape=jax.ShapeDtypeStruct((1, 128), jnp.float32),
        scratch_shapes=[pltpu.VMEM((_BSUB, _NPROD * 128), jnp.float32)],
    )(*([g2] * 9), _sel_matrix())
    pos_sum = jnp.sum(out[0, :12])
    neg_sum = jnp.sum(out[0, 12:_NCOL])
    return pos_sum / (_B * _L * _L) + neg_sum / (_B * _L * _NEG)


def kernel(batch, table):
    neg_words = jax.random.randint(
        jax.random.key(1), (_B, _NEG), 1, _VOCAB, dtype=jnp.int32)
    idx = jnp.concatenate([batch.T.reshape(-1), neg_words.T.reshape(-1)])
    idxq3 = jnp.where(idx < _HALF, idx, idx - _HALF).reshape(_NW, _NCH, _CH)
    hsel3 = (idx >= _HALF).astype(jnp.float32).reshape(_L + _NEG, 1, _B)
    table2 = _tc_squeeze(table.T)
    packed = _sc_gather(table2, idxq3)
    gathered = _tc_unpack(packed, hsel3)
    return _tc_loss(gathered)


# speedup vs baseline: 1.0406x; 1.0406x over previous
import functools

import jax
import jax.numpy as jnp
import numpy as np
from jax import lax
from jax.experimental import pallas as pl
from jax.experimental.pallas import tpu as pltpu
from jax.experimental.pallas import tpu_sc as plsc

_VOCAB = 1000000
_EMB = 64
_L = 50
_RAD = 5
_NEG = 5
_B = 4096

_NC = 2
_NS = 16
_NW = _NC * _NS
_POS_ROWS = _B * _L
_NEG_ROWS = _B * _NEG
_ROWS = _POS_ROWS + _NEG_ROWS
_RPW = _ROWS // _NW
_CH = 128
_NCH = _RPW // _CH

_BSUB = 2048
_NB2 = _B // _BSUB
_NPROD = 11
_HALF = 123 * 4096
_PACKED_ROWS = _POS_ROWS // 2 + _NEG_ROWS
_T = _L // 2


def _sc_gather(table2, idxq3):
    mesh = plsc.VectorSubcoreMesh(core_axis_name="c", subcore_axis_name="s")

    @functools.partial(
        pl.kernel,
        mesh=mesh,
        out_type=jax.ShapeDtypeStruct((_ROWS, 2 * _EMB), jnp.float32),
        scratch_types=[
            pltpu.VMEM((_NCH, _CH), jnp.int32),
            pltpu.VMEM((_CH, 2 * _EMB), jnp.float32),
            pltpu.VMEM((_CH, 2 * _EMB), jnp.float32),
            pltpu.SemaphoreType.DMA,
            pltpu.SemaphoreType.DMA,
        ],
    )
    def gather_kernel(table_hbm, idx_hbm, out_hbm, idx_v, rows_a, rows_b,
                      sem_a, sem_b):
        wid = lax.axis_index("s") * _NC + lax.axis_index("c")
        pltpu.sync_copy(idx_hbm.at[wid], idx_v)
        base = wid * _RPW

        def start(j, buf, sem):
            pltpu.async_copy(table_hbm.at[idx_v.at[j]], buf, sem)

        def finish(j, buf, sem):
            pltpu.make_async_copy(table_hbm.at[idx_v.at[j]], buf, sem).wait()
            pltpu.sync_copy(buf, out_hbm.at[pl.ds(base + j * _CH, _CH)])

        start(0, rows_a, sem_a)

        def body(m, carry):
            j0 = 2 * m
            start(j0 + 1, rows_b, sem_b)
            finish(j0, rows_a, sem_a)
            start(jnp.minimum(j0 + 2, _NCH - 1), rows_a, sem_a)
            finish(j0 + 1, rows_b, sem_b)
            return carry

        lax.fori_loop(0, _NCH // 2, body, 0)
        finish(_NCH - 1, rows_a, sem_a)

    return gather_kernel(table2, idxq3)


def _tc_squeeze(tT):
    bc = 12288
    grid = _HALF // bc

    def body(l_ref, r_ref, o_ref):
        o_ref[:, :_EMB] = jnp.transpose(l_ref[...], (1, 0))
        o_ref[:, _EMB:] = jnp.transpose(r_ref[...], (1, 0))

    return pl.pallas_call(
        body,
        grid=(grid,),
        in_specs=[
            pl.BlockSpec((_EMB, bc), lambda i: (0, i)),
            pl.BlockSpec((_EMB, bc),
                         lambda i: (0, jnp.minimum(i + grid,
                                                   _VOCAB // bc))),
        ],
        out_specs=pl.BlockSpec((bc, 2 * _EMB), lambda i: (i, 0)),
        out_shape=jax.ShapeDtypeStruct((_HALF, 2 * _EMB), jnp.float32),
    )(tT, tT)


def _tc_unpack(packed, hsel3):
    blk = _B
    npos = _L // 2

    def i1(o):
        return (jnp.where(o < npos, 2 * o, npos + o), 0)

    def i2(o):
        return (jnp.where(o < npos, 2 * o + 1, npos + o), 0)

    def body(p1_ref, p2_ref, h1_ref, h2_ref, o_ref):
        p1, p2 = p1_ref[...], p2_ref[...]
        h1 = jnp.transpose(h1_ref[...].reshape(1, blk), (1, 0))
        h2 = jnp.transpose(h2_ref[...].reshape(1, blk), (1, 0))
        o_ref[:, :_EMB] = jnp.where(h1 > 0, p1[:, _EMB:], p1[:, :_EMB])
        o_ref[:, _EMB:] = jnp.where(h2 > 0, p2[:, _EMB:], p2[:, :_EMB])

    return pl.pallas_call(
        body,
        grid=(_ROWS // blk // 2 + _NEG,),
        in_specs=[
            pl.BlockSpec((blk, 2 * _EMB), i1),
            pl.BlockSpec((blk, 2 * _EMB), i2),
            pl.BlockSpec((1, 1, blk), lambda o: (i1(o)[0], 0, 0)),
            pl.BlockSpec((1, 1, blk), lambda o: (i2(o)[0], 0, 0)),
        ],
        out_specs=pl.BlockSpec((blk, 2 * _EMB), lambda o: (o, 0)),
        out_shape=jax.ShapeDtypeStruct((_PACKED_ROWS, 2 * _EMB), jnp.float32),
    )(packed, packed, hsel3, hsel3)


def _sel_matrix():
    sel = np.zeros((_NPROD * 128, 128), np.float32)
    for p in range(_NPROD):
        sel[p * 128: p * 128 + 64, 2 * p] = 1.0
        sel[p * 128 + 64: (p + 1) * 128, 2 * p + 1] = 1.0
    return jnp.asarray(sel)


_POS_TMAX = [23, 23, 22, 22, 24, -1, 23, 23, 22, 22, -1, 21]
_NCOL = 2 * _NPROD


def _tc_loss(g2):
    negblk0 = _POS_ROWS // 2 // _BSUB

    def body(a_ref, b1_ref, b2_ref, b3_ref, n0, n1, n2, n3, n4, sel_ref,
             out_ref, s_ref):
        i2 = pl.program_id(0)
        t = pl.program_id(1)
        a = a_ref[...]
        prods = [
            a * b1_ref[...],
            a * b2_ref[...],
            a * pltpu.roll(a, 64, 1),
            a * pltpu.roll(b1_ref[...], 64, 1),
            a * pltpu.roll(b2_ref[...], 64, 1),
            a * pltpu.roll(b3_ref[...], 64, 1),
            a * n0[...],
            a * n1[...],
            a * n2[...],
            a * n3[...],
            a * n4[...],
        ]
        for p in range(_NPROD):
            s_ref[:, p * 128:(p + 1) * 128] = prods[p]
        sims = jnp.dot(s_ref[...], sel_ref[...],
                       preferred_element_type=jnp.float32)

        p_ = jax.nn.sigmoid(sims)
        f = jnp.where(p_ > 0, -jnp.log(jnp.where(p_ > 0, p_, 1.0)), 100.0)
        q_ = 1.0 - p_
        g = jnp.where(q_ > 0, -jnp.log(jnp.where(q_ > 0, q_, 1.0)), 100.0)

        lanes = lax.broadcasted_iota(jnp.int32, (1, 128), 1)
        tmax = jnp.full((1, 128), -1, jnp.int32)
        for c, tm in enumerate(_POS_TMAX):
            tmax = jnp.where(lanes == c, tm, tmax)
        is_pos = lanes < 12
        is_neg = (lanes >= 12) & (lanes < _NCOL)
        w = jnp.where(is_pos & (t <= tmax), 2.0,
                      jnp.where(is_neg, 1.0, 0.0))
        vals = jnp.where(is_pos, f, g) * w
        part = jnp.sum(vals, axis=0, keepdims=True)

        @pl.when((i2 == 0) & (t == 0))
        def _():
            out_ref[...] = jnp.zeros_like(out_ref)

        out_ref[...] += part

    bspec = lambda im: pl.BlockSpec((_BSUB, 128), im)
    out = pl.pallas_call(
        body,
        grid=(_NB2, _T),
        in_specs=[
            bspec(lambda i2, t: (t * _NB2 + i2, 0)),
            bspec(lambda i2, t: (jnp.minimum(t + 1, _T - 1) * _NB2 + i2, 0)),
            bspec(lambda i2, t: (jnp.minimum(t + 2, _T - 1) * _NB2 + i2, 0)),
            bspec(lambda i2, t: (jnp.minimum(t + 3, _T - 1) * _NB2 + i2, 0)),
            bspec(lambda i2, t: (negblk0 + 0 * _NB2 + i2, 0)),
            bspec(lambda i2, t: (negblk0 + 1 * _NB2 + i2, 0)),
            bspec(lambda i2, t: (negblk0 + 2 * _NB2 + i2, 0)),
            bspec(lambda i2, t: (negblk0 + 3 * _NB2 + i2, 0)),
            bspec(lambda i2, t: (negblk0 + 4 * _NB2 + i2, 0)),
            pl.BlockSpec((_NPROD * 128, 128), lambda i2, t: (0, 0)),
        ],
        out_specs=pl.BlockSpec((1, 128), lambda i2, t: (0, 0)),
        out_shape=jax.ShapeDtypeStruct((1, 128), jnp.float32),
        scratch_shapes=[pltpu.VMEM((_BSUB, _NPROD * 128), jnp.float32)],
    )(*([g2] * 9), _sel_matrix())
    pos_sum = jnp.sum(out[0, :12])
    neg_sum = jnp.sum(out[0, 12:_NCOL])
    return pos_sum / (_B * _L * _L) + neg_sum / (_B * _L * _NEG)


def kernel(batch, table):
    neg_words = jax.random.randint(
        jax.random.key(1), (_B, _NEG), 1, _VOCAB, dtype=jnp.int32)
    idx = jnp.concatenate([batch.T.reshape(-1), neg_words.T.reshape(-1)])
    idxq3 = jnp.where(idx < _HALF, idx, idx - _HALF).reshape(_NW, _NCH, _CH)
    hsel3 = (idx >= _HALF).astype(jnp.float32).reshape(_L + _NEG, 1, _B)
    table2 = _tc_squeeze(table.T)
    packed = _sc_gather(table2, idxq3)
    gathered = _tc_unpack(packed, hsel3)
    return _tc_loss(gathered)
